# baseline (speedup 1.0000x reference)
import functools

import jax
import jax.numpy as jnp
from jax import lax
from jax.experimental import pallas as pl
from jax.experimental.pallas import tpu as pltpu
from jax.experimental.pallas import tpu_sc as plsc


def _sum_body(nblk):
    def body(dist_ref, sum_ref):
        pid = pl.program_id(0)

        @pl.when(pid == 0)
        def _():
            sum_ref[...] = jnp.zeros_like(sum_ref)

        sum_ref[...] += dist_ref[...].sum(1)

    return body


def _epilogue_body(g_ref, sum_ref, lpw_ref, y_ref,
                   fp_ref, fd_ref, lq_ref, loss_ref):
    T, B = sum_ref.shape
    g = g_ref[...].reshape(T, B, B)
    lane = lax.broadcasted_iota(jnp.int32, g.shape, 2)
    bidx = lax.broadcasted_iota(jnp.int32, g.shape, 1)
    vals = jnp.where(lane == bidx, g, 0.0).sum(-1)
    probs = vals / sum_ref[...]
    fp_ref[...] = probs.T
    fd_ref[...] = probs[T - 1:T, :]
    lq = jnp.log(probs).sum(0, keepdims=True)
    lq_ref[...] = lq
    lp = (1.0 - y_ref[...]) * jnp.log(jnp.float32(1e-8)) + lpw_ref[...]
    d = lq - lp
    loss_ref[...] = jnp.mean(d * d).reshape(1, 1)


def _sc_gather(flat2d, rows):
    TB, B = rows.shape[0], flat2d.shape[1]
    n_workers = 32
    per_w = TB // n_workers
    mesh = plsc.VectorSubcoreMesh(core_axis_name="c", subcore_axis_name="s")

    @functools.partial(
        pl.kernel, mesh=mesh,
        out_type=jax.ShapeDtypeStruct((TB, B), jnp.float32),
        scratch_types=[
            pltpu.VMEM((per_w,), jnp.int32),
            pltpu.VMEM((per_w, B), jnp.float32),
            pltpu.SemaphoreType.DMA,
        ],
    )
    def gk(x_hbm, idx_hbm, out_hbm, idx_v, rows_v, sem):
        wid = lax.axis_index("s") * 2 + lax.axis_index("c")
        base = wid * per_w
        pltpu.sync_copy(idx_hbm.at[pl.ds(base, per_w)], idx_v)
        pltpu.async_copy(x_hbm.at[idx_v], rows_v, sem).wait()
        pltpu.sync_copy(rows_v, out_hbm.at[pl.ds(base, per_w)])

    return gk(flat2d, rows)


def kernel(distributions, actions, log_p_world, y):
    T, B, V = distributions.shape
    C = 5000
    nblk = V // C
    f32 = jnp.float32

    dvb = jnp.transpose(distributions, (0, 2, 1))
    flat2d = dvb.reshape(T * V, B)

    rows = (actions.astype(jnp.int32)
            + (jnp.arange(T, dtype=jnp.int32) * V)[:, None]).reshape(T * B)
    g = _sc_gather(flat2d, rows)

    sums = pl.pallas_call(
        _sum_body(nblk),
        grid=(nblk,),
        in_specs=[pl.BlockSpec((T, C, B), lambda i: (0, i, 0))],
        out_specs=pl.BlockSpec((T, B), lambda i: (0, 0)),
        out_shape=jax.ShapeDtypeStruct((T, B), f32),
        compiler_params=pltpu.CompilerParams(
            dimension_semantics=("arbitrary",),
        ),
    )(dvb)

    fp, fd, lq, loss = pl.pallas_call(
        _epilogue_body,
        out_shape=[
            jax.ShapeDtypeStruct((B, T), f32),
            jax.ShapeDtypeStruct((1, B), f32),
            jax.ShapeDtypeStruct((1, B), f32),
            jax.ShapeDtypeStruct((1, 1), f32),
        ],
    )(g, sums, log_p_world.reshape(1, B), y.reshape(1, B))

    return fp, fd.reshape(B), lq.reshape(B), loss[0, 0]
